# baseline (device time: 43741 ns/iter reference)
import jax
import jax.numpy as jnp
from jax import lax
from jax.experimental import pallas as pl
from jax.experimental.pallas import tpu as pltpu

N_DEV = 32
GROUP = 8


def kernel(x, Win0, Wout0, Win1, Wout1, Win2, Wout2):
    B, D = x.shape
    rows = B // N_DEV

    def body(x_ref, win0, wout0, win1, wout1, win2, wout2, out_ref,
             xg_ref, p_ref, red_ref, rs_ref,
             s1_sems, s2_sems, p1_sems, p2_sems, local_sem):
        me = lax.axis_index("i")

        barrier = pltpu.get_barrier_semaphore()
        for d in range(1, N_DEV):
            pl.semaphore_signal(
                barrier, inc=1,
                device_id=((me + d) % N_DEV,),
                device_id_type=pl.DeviceIdType.MESH,
            )
        pl.semaphore_wait(barrier, N_DEV - 1)

        def p1_send(j):
            dst = (me + j) % N_DEV
            rdma = pltpu.make_async_remote_copy(
                src_ref=p_ref.at[j],
                dst_ref=rs_ref.at[N_DEV - j],
                send_sem=s1_sems.at[j - 1],
                recv_sem=p1_sems.at[N_DEV - j],
                device_id=(dst,),
                device_id_type=pl.DeviceIdType.MESH,
            )
            rdma.start()
            return rdma

        def p2_send(d):
            dst = (me + d) % N_DEV
            rdma = pltpu.make_async_remote_copy(
                src_ref=red_ref,
                dst_ref=xg_ref.at[N_DEV - d],
                send_sem=s2_sems.at[d - 1],
                recv_sem=p2_sems.at[N_DEV - d],
                device_id=(dst,),
                device_id_type=pl.DeviceIdType.MESH,
            )
            rdma.start()
            return rdma

        def reduce_scatter_finish(sends1):
            for r in sends1:
                r.wait_recv()
            red = jnp.sum(rs_ref[:].astype(jnp.float32), axis=0)
            for r in sends1:
                r.wait_send()
            return red

        def all_gather_start(red):
            red_ref[:] = red.astype(jnp.bfloat16)
            xg_ref[0] = red_ref[:]
            return [p2_send(d) for d in range(1, N_DEV)]

        def mlp(xb, win, wout):
            h = jnp.dot(xb, win[:], preferred_element_type=jnp.float32)
            h = jnp.maximum(h, 0.0).astype(jnp.bfloat16)
            return jnp.dot(
                h, wout[:], preferred_element_type=jnp.float32
            ).astype(jnp.bfloat16)

        p_ref[:] = mlp(x_ref[:].astype(jnp.bfloat16), win0, wout0).reshape(
            N_DEV, rows, D)
        sends1 = []
        for d in range(1, N_DEV):
            dst = (me + d) % N_DEV
            rdma = pltpu.make_async_remote_copy(
                src_ref=p_ref.at[dst],
                dst_ref=rs_ref.at[N_DEV - d],
                send_sem=s1_sems.at[d - 1],
                recv_sem=p1_sems.at[N_DEV - d],
                device_id=(dst,),
                device_id_type=pl.DeviceIdType.MESH,
            )
            rdma.start()
            sends1.append(rdma)
        own = pltpu.make_async_copy(p_ref.at[me], rs_ref.at[0], local_sem)
        own.start()
        own.wait()
        red = reduce_scatter_finish(sends1)
        sends2 = all_gather_start(red)

        for layer_i, (win, wout) in enumerate([(win1, wout1), (win2, wout2)]):
            last = layer_i == 1
            sends1 = []
            for g in range(N_DEV // GROUP):
                for k in range(max(g * GROUP, 1), (g + 1) * GROUP):
                    sends2[N_DEV - 1 - k].wait_recv()
                xb = xg_ref[pl.ds(g * GROUP, GROUP)].reshape(GROUP * rows, D)
                p_ref[pl.ds(g * GROUP, GROUP)] = mlp(xb, win, wout).reshape(
                    GROUP, rows, D)
                for k in range(max(g * GROUP, 1), (g + 1) * GROUP):
                    sends1.append(p1_send(k))
            rs_ref[0] = p_ref[0]
            for r in sends2:
                r.wait_send()
            red = reduce_scatter_finish(sends1)
            if last:
                out_ref[:] = red
            else:
                sends2 = all_gather_start(red)

    return pl.pallas_call(
        body,
        out_shape=jax.ShapeDtypeStruct((rows, D), jnp.float32),
        in_specs=[pl.BlockSpec(memory_space=pltpu.VMEM)] * 7,
        out_specs=pl.BlockSpec(memory_space=pltpu.VMEM),
        scratch_shapes=[
            pltpu.VMEM((N_DEV, rows, D), jnp.bfloat16),
            pltpu.VMEM((N_DEV, rows, D), jnp.bfloat16),
            pltpu.VMEM((rows, D), jnp.bfloat16),
            pltpu.VMEM((N_DEV, rows, D), jnp.bfloat16),
            pltpu.SemaphoreType.DMA((N_DEV - 1,)),
            pltpu.SemaphoreType.DMA((N_DEV - 1,)),
            pltpu.SemaphoreType.DMA((N_DEV,)),
            pltpu.SemaphoreType.DMA((N_DEV,)),
            pltpu.SemaphoreType.DMA,
        ],
        compiler_params=pltpu.CompilerParams(collective_id=0),
    )(x, Win0, Wout0, Win1, Wout1, Win2, Wout2)
